# baseline (speedup 1.0000x reference)
import jax
import jax.numpy as jnp
from jax import lax
from jax.experimental import pallas as pl
from jax.experimental.pallas import tpu as pltpu
from jax.experimental.pallas import tpu_sc as plsc

B = 16384
L = 20
D = 64
V = 1000000
LANES = 16
NC, NS = 2, 16
NW = NC * NS
BPW = B // NW
KD = D // LANES

CT = 32768
HCT = CT // 2
NGRID = (V + CT - 1) // CT
NPAIR = NGRID * HCT

CB = 8
NCHUNK = BPW // CB
IDX_PER_CHUNK = CB * L
STREAMS = (80, 80)


def _transpose_body(src, dst):
    t = src[...].T
    dst[...] = jnp.concatenate([t[0:HCT], t[HCT:CT]], axis=1)


def _pair_ids(v):
    return jnp.bitwise_or(
        lax.shift_left(lax.shift_right_logical(v, 15), 14),
        jnp.bitwise_and(v, HCT - 1))


def _lookup_body(os_hbm, nodes_hbm, table_hbm, y_hbm,
                 idx_v, blk_v, nodes_v, nblk_v, g_v, nrows_v, out_v, sems):
    wid = lax.axis_index("s") * NC + lax.axis_index("c")
    base = wid * BPW

    pltpu.sync_copy(os_hbm.at[pl.ds(base * L, BPW * L)],
                    idx_v.at[pl.ds(0, BPW * L)])
    pltpu.sync_copy(nodes_hbm.at[pl.ds(base, BPW)],
                    nodes_v.at[pl.ds(0, BPW)])

    def blk_body(i, carry):
        off = pl.multiple_of(i * LANES, LANES)
        blk_v[pl.ds(off, LANES)] = _pair_ids(idx_v[pl.ds(off, LANES)])
        return carry

    lax.fori_loop(0, BPW * L // LANES, blk_body, 0)

    def nblk_body(i, carry):
        off = pl.multiple_of(i * LANES, LANES)
        nblk_v[pl.ds(off, LANES)] = _pair_ids(nodes_v[pl.ds(off, LANES)])
        return carry

    lax.fori_loop(0, BPW // LANES, nblk_body, 0)

    lane_ids = jnp.arange(LANES, dtype=jnp.int32)

    def shuffle(v, idx):
        return v.at[idx].get(mode="promise_in_bounds")

    def lane_sum(v):
        for sh in (8, 4, 2, 1):
            v = v + shuffle(v, lane_ids ^ sh)
        return v

    def half_bit(v):
        return jnp.bitwise_and(lax.shift_right_logical(v, 14),
                               1).astype(jnp.float32)

    def dmas(c, buf):
        cw = jnp.minimum(c, NCHUNK - 1)
        coff = pl.multiple_of(cw * IDX_PER_CHUNK, 8)
        goff = pl.multiple_of(cw * CB, 8)
        out = [pltpu.make_async_copy(
            table_hbm.at[nblk_v.at[pl.ds(goff, CB)]],
            nrows_v.at[buf], sems.at[2 + buf])]
        off = 0
        for n in STREAMS:
            out.append(pltpu.make_async_copy(
                table_hbm.at[blk_v.at[pl.ds(coff + off, n)]],
                g_v.at[buf, pl.ds(off, n)], sems.at[buf]))
            off += n
        return out

    def issue(c, buf):
        for d in dmas(c, buf):
            d.start()

    def wait(c, buf):
        for d in dmas(c, buf):
            d.wait()

    def compute(c, buf, loff, vec0):
        goff = pl.multiple_of(c * CB, 8)
        nhods = half_bit(nodes_v[pl.ds(goff, LANES)])

        def lane_body(lane, vec):
            b = c * CB + lane
            woff = jnp.bitwise_and(b, 1) * 4
            a0 = pl.multiple_of(b * L - woff, 8)
            v0 = idx_v[pl.ds(a0, LANES)]
            v1 = idx_v[pl.ds(a0 + LANES, LANES)]
            si = lane_ids + woff
            s15 = jnp.bitwise_and(si, 15)
            g0 = shuffle(v0, s15)
            g1 = shuffle(v1, s15)
            hib = lax.shift_right_logical(si, 4)
            iv0 = g0 + (g1 - g0) * hib
            iv1 = shuffle(v1, s15)
            h0 = half_bit(iv0)
            h1 = half_bit(iv1)
            t = jnp.zeros((LANES,), jnp.float32)
            accs = [jnp.zeros((LANES,), jnp.float32) for _ in range(KD)]
            for j in range(L):
                hsrc, jj = (h0, j) if j < LANES else (h1, j - LANES)
                hf = shuffle(hsrc, jnp.full((LANES,), jj, jnp.int32))
                row = lane * L + j
                for k in range(KD):
                    lo = g_v[buf, row, pl.ds(k * LANES, LANES)]
                    hi = g_v[buf, row, pl.ds(D + k * LANES, LANES)]
                    accs[k] = accs[k] + (lo + (hi - lo) * hf)
            nhf = shuffle(nhods, jnp.full((LANES,), lane, jnp.int32))
            for k in range(KD):
                nlo = nrows_v[buf, lane, pl.ds(k * LANES, LANES)]
                nhi = nrows_v[buf, lane, pl.ds(D + k * LANES, LANES)]
                t = t + accs[k] * (nlo + (nhi - nlo) * nhf)
            s = lane_sum(t) * (1.0 / L)
            return jnp.where(lane_ids == loff + lane, s, vec)

        return lax.fori_loop(0, CB, lane_body, vec0)

    issue(jnp.int32(0), 0)
    issue(jnp.int32(1), 1)

    def pair_body(i, carry):
        ca = 2 * i
        wait(ca, 0)
        vec = compute(ca, 0, 0, jnp.zeros((LANES,), jnp.float32))
        issue(ca + 2, 0)
        wait(ca + 1, 1)
        vec = compute(ca + 1, 1, CB, vec)
        issue(ca + 3, 1)
        out_v[pl.ds(pl.multiple_of(i * LANES, LANES), LANES)] = vec
        return carry

    lax.fori_loop(0, NCHUNK // 2, pair_body, 0)
    wait(jnp.int32(NCHUNK - 1), 0)
    wait(jnp.int32(NCHUNK - 1), 1)

    def sig_body(i, carry):
        off = pl.multiple_of(i * LANES, LANES)
        v = out_v[pl.ds(off, LANES)]
        out_v[pl.ds(off, LANES)] = 1.0 / (1.0 + jnp.exp(-v))
        return carry

    lax.fori_loop(0, BPW // LANES, sig_body, 0)
    pltpu.sync_copy(out_v, y_hbm.at[pl.ds(base, BPW)])


def kernel(os, nodes, node_embs):
    os_flat = os.reshape(-1)
    table_t = node_embs.T

    paired = pl.pallas_call(
        _transpose_body,
        grid=(NGRID,),
        in_specs=[pl.BlockSpec((D, CT), lambda p: (0, p))],
        out_specs=pl.BlockSpec((HCT, 2 * D), lambda p: (p, 0)),
        out_shape=jax.ShapeDtypeStruct((NPAIR, 2 * D), jnp.float32),
    )(table_t)

    mesh = plsc.VectorSubcoreMesh(core_axis_name="c", subcore_axis_name="s")
    lookup = pl.kernel(
        _lookup_body,
        mesh=mesh,
        out_type=jax.ShapeDtypeStruct((B,), jnp.float32),
        scratch_types=[
            pltpu.VMEM((BPW * L + 2 * LANES,), jnp.int32),
            pltpu.VMEM((BPW * L,), jnp.int32),
            pltpu.VMEM((BPW + LANES,), jnp.int32),
            pltpu.VMEM((BPW,), jnp.int32),
            pltpu.VMEM((2, IDX_PER_CHUNK, 2 * D), jnp.float32),
            pltpu.VMEM((2, CB, 2 * D), jnp.float32),
            pltpu.VMEM((BPW,), jnp.float32),
            pltpu.SemaphoreType.DMA((4,)),
        ],
    )
    return lookup(os_flat, nodes, paired)


# speedup vs baseline: 1.8149x; 1.8149x over previous
import jax
import jax.numpy as jnp
from jax import lax
from jax.experimental import pallas as pl
from jax.experimental.pallas import tpu as pltpu
from jax.experimental.pallas import tpu_sc as plsc

B = 16384
L = 20
D = 64
V = 1000000
LANES = 16
NC, NS = 2, 16
NW = NC * NS
BPW = B // NW
KD = D // LANES

CT = 32768
NGRID = (V + CT - 1) // CT

CB = 16
NCHUNK = BPW // CB
IDX_PER_CHUNK = CB * L
STREAMS = (128, 128, 64)


def _transpose_body(src, dst):
    dst[:, 0:D] = src[...].T


def _lookup_body(os_hbm, nodes_hbm, table_hbm, y_hbm,
                 idx_v, nodes_v, g_v, nrows_v, out_v, sems):
    wid = lax.axis_index("s") * NC + lax.axis_index("c")
    base = wid * BPW

    pltpu.sync_copy(os_hbm.at[pl.ds(base * L, BPW * L)], idx_v)
    pltpu.sync_copy(nodes_hbm.at[pl.ds(base, BPW)], nodes_v)

    lane_ids = jnp.arange(LANES, dtype=jnp.int32)

    def shuffle(v, idx):
        return v.at[idx].get(mode="promise_in_bounds")

    def lane_sum(v):
        for sh in (8, 4, 2, 1):
            v = v + shuffle(v, lane_ids ^ sh)
        return v

    def issue(c, buf):
        cw = jnp.minimum(c, NCHUNK - 1)
        coff = pl.multiple_of(cw * IDX_PER_CHUNK, 8)
        goff = pl.multiple_of(cw * CB, 8)
        copies = [pltpu.async_copy(
            table_hbm.at[nodes_v.at[pl.ds(goff, CB)]],
            nrows_v.at[buf], sems.at[2 + buf])]
        off = 0
        for n in STREAMS:
            copies.append(pltpu.async_copy(
                table_hbm.at[idx_v.at[pl.ds(coff + off, n)]],
                g_v.at[buf, pl.ds(off, n)], sems.at[buf]))
            off += n
        return copies

    def compute(c, buf):
        def lane_body(lane, vec):
            t = jnp.zeros((LANES,), jnp.float32)
            for k in range(KD):
                col = pl.ds(k * LANES, LANES)
                acc = g_v[buf, lane * L, col]
                for j in range(1, L):
                    acc = acc + g_v[buf, lane * L + j, col]
                t = t + acc * nrows_v[buf, lane, col]
            s = lane_sum(t) * (1.0 / L)
            return jnp.where(lane_ids == lane, s, vec)

        vec = lax.fori_loop(0, CB, lane_body, jnp.zeros((LANES,),
                                                        jnp.float32))
        out_v[pl.ds(pl.multiple_of(c * CB, LANES), LANES)] = vec

    prime = issue(jnp.int32(0), 0)

    def pair_body(i, carry):
        ca, cb2 = 2 * i, 2 * i + 1
        pend_a = carry
        pend_b = issue(cb2, 1)
        for cp in prime if pend_a is None else pend_a:
            cp.wait()
        compute(ca, 0)
        pend_a2 = issue(ca + 2, 0)
        for cp in pend_b:
            cp.wait()
        compute(cb2, 1)
        return pend_a2

    pend_a = None
    for i in range(NCHUNK // 2):
        pend_a = pair_body(i, pend_a)
    for cp in pend_a:
        cp.wait()

    def sig_body(i, carry):
        off = pl.multiple_of(i * LANES, LANES)
        v = out_v[pl.ds(off, LANES)]
        out_v[pl.ds(off, LANES)] = 1.0 / (1.0 + jnp.exp(-v))
        return carry

    lax.fori_loop(0, BPW // LANES, sig_body, 0)
    pltpu.sync_copy(out_v, y_hbm.at[pl.ds(base, BPW)])


def kernel(os, nodes, node_embs):
    os_flat = os.reshape(-1)
    table_t = node_embs.T

    wide = pl.pallas_call(
        _transpose_body,
        grid=(NGRID,),
        in_specs=[pl.BlockSpec((D, CT), lambda p: (0, p))],
        out_specs=pl.BlockSpec((CT, 2 * D), lambda p: (p, 0)),
        out_shape=jax.ShapeDtypeStruct((V, 2 * D), jnp.float32),
    )(table_t)

    mesh = plsc.VectorSubcoreMesh(core_axis_name="c", subcore_axis_name="s")
    lookup = pl.kernel(
        _lookup_body,
        mesh=mesh,
        out_type=jax.ShapeDtypeStruct((B,), jnp.float32),
        scratch_types=[
            pltpu.VMEM((BPW * L,), jnp.int32),
            pltpu.VMEM((BPW,), jnp.int32),
            pltpu.VMEM((2, IDX_PER_CHUNK, 2 * D), jnp.float32),
            pltpu.VMEM((2, CB, 2 * D), jnp.float32),
            pltpu.VMEM((BPW,), jnp.float32),
            pltpu.SemaphoreType.DMA((4,)),
        ],
    )
    return lookup(os_flat, nodes, wide)
